# baseline (speedup 1.0000x reference)
import functools

import jax
import jax.numpy as jnp
from jax import lax
from jax.experimental import pallas as pl
from jax.experimental.pallas import tpu as pltpu
from jax.experimental.pallas import tpu_sc as plsc

N_NODES = 10000
N_EDGES = 320000
DIM = 128
NP = 10240
CHUNK = 128
NC, NS = 2, 16
NW = NC * NS
N_CHUNKS = N_EDGES // CHUNK
UNROLL = 6
NROWS = 2
NIDX = 6
NITER = 13


def _sc_aggregate(seq, edge_flat):
    mesh = plsc.VectorSubcoreMesh(core_axis_name="c", subcore_axis_name="s")

    scratch = [
        pltpu.VMEM_SHARED((NP, DIM), jnp.float32),
        pltpu.VMEM_SHARED((NP,), jnp.float32),
        pltpu.VMEM((CHUNK,), jnp.float32),
        pltpu.VMEM((640,), jnp.float32),
        pltpu.VMEM((64, DIM), jnp.float32),
        pltpu.VMEM((8, DIM), jnp.float32),
    ]
    scratch += [pltpu.VMEM((CHUNK,), jnp.int32) for _ in range(NIDX)]
    scratch += [pltpu.VMEM((CHUNK,), jnp.int32) for _ in range(NIDX)]
    scratch += [pltpu.VMEM((CHUNK, DIM), jnp.float32) for _ in range(NROWS)]
    scratch += [pltpu.SemaphoreType.DMA for _ in range(2 * NIDX + NROWS)]
    scratch += [pltpu.SemaphoreType.DMA]
    scratch += [pltpu.SemaphoreType.DMA for _ in range(NROWS)]

    @functools.partial(
        pl.kernel,
        out_type=(
            jax.ShapeDtypeStruct((NC, NP, DIM), jnp.float32),
            jax.ShapeDtypeStruct((NC, 80, DIM), jnp.float32),
        ),
        mesh=mesh,
        scratch_types=scratch,
    )
    def sc_agg(seq_hbm, edge_hbm, part_hbm, degp_hbm,
               acc_sh, deg_sh, ones_v, zero1_v, zero_v, dbounce_v, *bufs):
        src_vs = bufs[0:NIDX]
        dst_vs = bufs[NIDX:2 * NIDX]
        rows_vs = bufs[2 * NIDX:2 * NIDX + NROWS]
        isems = bufs[2 * NIDX + NROWS:3 * NIDX + NROWS]
        gsems = bufs[3 * NIDX + NROWS:3 * NIDX + 2 * NROWS]
        dsems = bufs[3 * NIDX + 2 * NROWS:4 * NIDX + 2 * NROWS]
        zsem = bufs[4 * NIDX + 2 * NROWS]
        ssems = bufs[4 * NIDX + 2 * NROWS + 1:4 * NIDX + 3 * NROWS + 1]

        cid = lax.axis_index("c")
        sid = lax.axis_index("s")
        wid = cid * NS + sid
        zvec = jnp.zeros((16,), jnp.float32)

        def zb(i, _):
            zero_v[i // 8, pl.ds((i % 8) * 16, 16)] = zvec
            return 0
        lax.fori_loop(0, 64 * 8, zb, 0)

        def z1(i, _):
            zero1_v[pl.ds(i * 16, 16)] = zvec
            return 0
        lax.fori_loop(0, 40, z1, 0)

        for j in range(CHUNK // 16):
            ones_v[pl.ds(j * 16, 16)] = jnp.ones((16,), jnp.float32)

        zdescs = [
            pltpu.async_copy(zero_v, acc_sh.at[pl.ds(sid * 640 + t * 64, 64)],
                             zsem)
            for t in range(10)
        ]
        zdescs.append(
            pltpu.async_copy(zero1_v, deg_sh.at[pl.ds(sid * 640, 640)], zsem))

        N_MAIN = NITER * UNROLL

        def issue_idx(v, s):
            base = pl.multiple_of((wid + v * NW) * CHUNK, CHUNK)
            pltpu.async_copy(edge_hbm.at[pl.ds(N_EDGES + base, CHUNK)],
                             src_vs[s], isems[s])
            pltpu.async_copy(edge_hbm.at[pl.ds(base, CHUNK)],
                             dst_vs[s], isems[s])

        def wait_idx(s):
            pltpu.make_async_copy(edge_hbm.at[pl.ds(0, CHUNK)], src_vs[s],
                                  isems[s]).wait()
            pltpu.make_async_copy(edge_hbm.at[pl.ds(0, CHUNK)], dst_vs[s],
                                  isems[s]).wait()

        def issue_gather(s, b):
            pltpu.async_copy(seq_hbm.at[src_vs[s]], rows_vs[b], gsems[b])

        def wait_gather(s, b):
            pltpu.make_async_copy(seq_hbm.at[src_vs[s]], rows_vs[b],
                                  gsems[b]).wait()

        def wait_deg(s):
            pltpu.make_async_copy(ones_v, deg_sh.at[dst_vs[s]],
                                  dsems[s]).wait()

        def wait_scatter(b):
            pltpu.make_async_copy(rows_vs[b], acc_sh.at[dst_vs[0]],
                                  ssems[b]).wait()

        for v in range(4):
            issue_idx(v, v % NIDX)
        wait_idx(0)
        issue_gather(0, 0)

        for zd in zdescs:
            zd.wait()
        plsc.subcore_barrier()

        def body(t, _):
            c0 = t * UNROLL
            for u in range(UNROLL):
                b = u % NROWS
                bp = (u + 1) % NROWS
                wait_gather(u, b)
                pltpu.async_copy(rows_vs[b], acc_sh.at[dst_vs[u]], ssems[b],
                                 add=True)
                pltpu.async_copy(ones_v, deg_sh.at[dst_vs[u]], dsems[u],
                                 add=True)

                @pl.when(c0 + u >= 2)
                def _(u=u):
                    wait_deg((u - 2) % NIDX)

                @pl.when(c0 + u + 4 < N_MAIN)
                def _(u=u, c0=c0):
                    issue_idx(c0 + u + 4, (u + 4) % NIDX)

                @pl.when(c0 + u >= 1)
                def _(bp=bp):
                    wait_scatter(bp)

                @pl.when(c0 + u + 1 < N_MAIN)
                def _(u=u, bp=bp):
                    s = (u + 1) % NIDX
                    wait_idx(s)
                    issue_gather(s, bp)
            return 0
        lax.fori_loop(0, NITER, body, 0)

        wait_scatter((N_MAIN - 1) % NROWS)
        wait_deg((N_MAIN - 2) % NIDX)
        wait_deg((N_MAIN - 1) % NIDX)

        @pl.when(wid < N_CHUNKS - NITER * UNROLL * NW)
        def _():
            base = pl.multiple_of((NITER * UNROLL * NW + wid) * CHUNK, CHUNK)
            pltpu.sync_copy(edge_hbm.at[pl.ds(N_EDGES + base, CHUNK)],
                            src_vs[0])
            pltpu.sync_copy(edge_hbm.at[pl.ds(base, CHUNK)], dst_vs[0])
            pltpu.async_copy(seq_hbm.at[src_vs[0]], rows_vs[0],
                             gsems[0]).wait()
            pltpu.sync_copy(rows_vs[0], acc_sh.at[dst_vs[0]], add=True)
            pltpu.sync_copy(ones_v, deg_sh.at[dst_vs[0]], add=True)

        plsc.subcore_barrier()

        cdesc = pltpu.async_copy(acc_sh.at[pl.ds(sid * 640, 640)],
                                 part_hbm.at[cid, pl.ds(sid * 640, 640)],
                                 zsem)

        @pl.when(sid < 10)
        def _():
            bdescs = [
                pltpu.async_copy(
                    deg_sh.at[pl.ds((sid * 8 + q) * DIM, DIM)],
                    dbounce_v.at[q], zsem)
                for q in range(8)
            ]
            for bd in bdescs:
                bd.wait()
            pltpu.sync_copy(dbounce_v, degp_hbm.at[cid, pl.ds(sid * 8, 8)])

        cdesc.wait()

    return sc_agg(seq, edge_flat)


def _tc_deg_reduce(degp):
    def body(d_r, o_r):
        o_r[...] = 1.0 / (d_r[0] + d_r[1] + 1e-8)

    return pl.pallas_call(
        body,
        out_shape=jax.ShapeDtypeStruct((80, DIM), jnp.float32),
    )(degp)


def _tc_finish(part, dinv, W, a1):
    BR = 1000
    grid = (N_NODES // BR,)

    def body(p0_r, p1_r, d_r, w_r, a_r, o_r):
        x = p0_r[0] + p1_r[0]
        y = lax.dot_general(x, w_r[...], (((1,), (1,)), ((), ())),
                            preferred_element_type=jnp.float32)
        y = y * d_r[...]
        a = a_r[0]
        o_r[...] = jnp.where(y >= 0.0, y, a * y)

    return pl.pallas_call(
        body,
        grid=grid,
        in_specs=[
            pl.BlockSpec((1, BR, DIM), lambda i: (0, i, 0)),
            pl.BlockSpec((1, BR, DIM), lambda i: (1, i, 0)),
            pl.BlockSpec((BR, 1), lambda i: (i, 0)),
            pl.BlockSpec((DIM, DIM), lambda i: (0, 0)),
            pl.BlockSpec(memory_space=pltpu.SMEM),
        ],
        out_specs=pl.BlockSpec((BR, DIM), lambda i: (i, 0)),
        out_shape=jax.ShapeDtypeStruct((N_NODES, DIM), jnp.float32),
    )(part, part, dinv, W, a1)


def kernel(seq, edge_index, W, prelu_a):
    edge_flat = edge_index.astype(jnp.int32).reshape(2 * N_EDGES)
    part, degp = _sc_aggregate(seq, edge_flat)
    dinv = _tc_deg_reduce(degp).reshape(NP)[:N_NODES].reshape(N_NODES, 1)
    a1 = prelu_a.reshape(1)
    return _tc_finish(part, dinv, W, a1)


# speedup vs baseline: 1.1553x; 1.1553x over previous
import functools

import jax
import jax.numpy as jnp
from jax import lax
from jax.experimental import pallas as pl
from jax.experimental.pallas import tpu as pltpu
from jax.experimental.pallas import tpu_sc as plsc

N_NODES = 10000
N_EDGES = 320000
DIM = 128
NP = 10240
CHUNK = 128
NC, NS = 2, 16
NW = NC * NS
N_CHUNKS = N_EDGES // CHUNK
UNROLL = 6
NROWS = 2
NIDX = 6
NITER = 13


def _sc_aggregate(seq, edge_flat):
    mesh = plsc.VectorSubcoreMesh(core_axis_name="c", subcore_axis_name="s")

    scratch = [
        pltpu.VMEM_SHARED((NP, DIM), jnp.float32),
        pltpu.VMEM_SHARED((NP,), jnp.float32),
        pltpu.VMEM((CHUNK,), jnp.float32),
        pltpu.VMEM((640,), jnp.float32),
        pltpu.VMEM((64, DIM), jnp.float32),
        pltpu.VMEM((8, DIM), jnp.float32),
    ]
    scratch += [pltpu.VMEM((CHUNK,), jnp.int32) for _ in range(NIDX)]
    scratch += [pltpu.VMEM((CHUNK,), jnp.int32) for _ in range(NIDX)]
    scratch += [pltpu.VMEM((CHUNK, DIM), jnp.float32) for _ in range(NROWS)]
    scratch += [pltpu.SemaphoreType.DMA for _ in range(2 * NIDX + NROWS)]
    scratch += [pltpu.SemaphoreType.DMA]

    @functools.partial(
        pl.kernel,
        out_type=(
            jax.ShapeDtypeStruct((NC, NP, DIM), jnp.float32),
            jax.ShapeDtypeStruct((NC, 80, DIM), jnp.float32),
        ),
        mesh=mesh,
        scratch_types=scratch,
    )
    def sc_agg(seq_hbm, edge_hbm, part_hbm, degp_hbm,
               acc_sh, deg_sh, ones_v, zero1_v, zero_v, dbounce_v, *bufs):
        src_vs = bufs[0:NIDX]
        dst_vs = bufs[NIDX:2 * NIDX]
        rows_vs = bufs[2 * NIDX:2 * NIDX + NROWS]
        isems = bufs[2 * NIDX + NROWS:3 * NIDX + NROWS]
        gsems = bufs[3 * NIDX + NROWS:3 * NIDX + 2 * NROWS]
        dsems = bufs[3 * NIDX + 2 * NROWS:4 * NIDX + 2 * NROWS]
        zsem = bufs[4 * NIDX + 2 * NROWS]

        cid = lax.axis_index("c")
        sid = lax.axis_index("s")
        wid = cid * NS + sid
        zvec = jnp.zeros((16,), jnp.float32)

        def zb(i, _):
            zero_v[i // 8, pl.ds((i % 8) * 16, 16)] = zvec
            return 0
        lax.fori_loop(0, 64 * 8, zb, 0)

        def z1(i, _):
            zero1_v[pl.ds(i * 16, 16)] = zvec
            return 0
        lax.fori_loop(0, 40, z1, 0)

        for j in range(CHUNK // 16):
            ones_v[pl.ds(j * 16, 16)] = jnp.ones((16,), jnp.float32)

        zdescs = [
            pltpu.async_copy(zero_v, acc_sh.at[pl.ds(sid * 640 + t * 64, 64)],
                             zsem)
            for t in range(10)
        ]
        zdescs.append(
            pltpu.async_copy(zero1_v, deg_sh.at[pl.ds(sid * 640, 640)], zsem))

        N_MAIN = NITER * UNROLL

        def issue_idx(v, s):
            base = pl.multiple_of((wid + v * NW) * CHUNK, CHUNK)
            pltpu.async_copy(edge_hbm.at[pl.ds(N_EDGES + base, CHUNK)],
                             src_vs[s], isems[s])
            pltpu.async_copy(edge_hbm.at[pl.ds(base, CHUNK)],
                             dst_vs[s], isems[s])

        def wait_idx(s):
            pltpu.make_async_copy(edge_hbm.at[pl.ds(0, CHUNK)], src_vs[s],
                                  isems[s]).wait()
            pltpu.make_async_copy(edge_hbm.at[pl.ds(0, CHUNK)], dst_vs[s],
                                  isems[s]).wait()

        def issue_gather(s, b):
            pltpu.async_copy(seq_hbm.at[src_vs[s]], rows_vs[b], gsems[b])

        def wait_gather(s, b):
            pltpu.make_async_copy(seq_hbm.at[src_vs[s]], rows_vs[b],
                                  gsems[b]).wait()

        def wait_deg(s):
            pltpu.make_async_copy(ones_v, deg_sh.at[dst_vs[s]],
                                  dsems[s]).wait()

        for v in range(4):
            issue_idx(v, v % NIDX)
        wait_idx(0)
        issue_gather(0, 0)
        wait_idx(1)
        issue_gather(1, 1)

        for zd in zdescs:
            zd.wait()
        plsc.subcore_barrier()

        def body(t, _):
            c0 = t * UNROLL
            for u in range(UNROLL):
                b = u % NROWS
                wait_gather(u, b)
                pltpu.sync_copy(rows_vs[b], acc_sh.at[dst_vs[u]], add=True)
                pltpu.async_copy(ones_v, deg_sh.at[dst_vs[u]], dsems[u],
                                 add=True)

                @pl.when(c0 + u >= 2)
                def _(u=u):
                    wait_deg((u - 2) % NIDX)

                @pl.when(c0 + u + 4 < N_MAIN)
                def _(u=u, c0=c0):
                    issue_idx(c0 + u + 4, (u + 4) % NIDX)

                @pl.when(c0 + u + 2 < N_MAIN)
                def _(u=u, b=b):
                    s = (u + 2) % NIDX
                    wait_idx(s)
                    issue_gather(s, b)
            return 0
        lax.fori_loop(0, NITER, body, 0)

        wait_deg((N_MAIN - 2) % NIDX)
        wait_deg((N_MAIN - 1) % NIDX)

        @pl.when(wid < N_CHUNKS - NITER * UNROLL * NW)
        def _():
            base = pl.multiple_of((NITER * UNROLL * NW + wid) * CHUNK, CHUNK)
            pltpu.sync_copy(edge_hbm.at[pl.ds(N_EDGES + base, CHUNK)],
                            src_vs[0])
            pltpu.sync_copy(edge_hbm.at[pl.ds(base, CHUNK)], dst_vs[0])
            pltpu.async_copy(seq_hbm.at[src_vs[0]], rows_vs[0],
                             gsems[0]).wait()
            pltpu.sync_copy(rows_vs[0], acc_sh.at[dst_vs[0]], add=True)
            pltpu.sync_copy(ones_v, deg_sh.at[dst_vs[0]], add=True)

        plsc.subcore_barrier()

        cdesc = pltpu.async_copy(acc_sh.at[pl.ds(sid * 640, 640)],
                                 part_hbm.at[cid, pl.ds(sid * 640, 640)],
                                 zsem)

        @pl.when(sid < 10)
        def _():
            bdescs = [
                pltpu.async_copy(
                    deg_sh.at[pl.ds((sid * 8 + q) * DIM, DIM)],
                    dbounce_v.at[q], zsem)
                for q in range(8)
            ]
            for bd in bdescs:
                bd.wait()
            pltpu.sync_copy(dbounce_v, degp_hbm.at[cid, pl.ds(sid * 8, 8)])

        cdesc.wait()

    return sc_agg(seq, edge_flat)


def _tc_deg_reduce(degp):
    def body(d_r, o_r):
        o_r[...] = 1.0 / (d_r[0] + d_r[1] + 1e-8)

    return pl.pallas_call(
        body,
        out_shape=jax.ShapeDtypeStruct((80, DIM), jnp.float32),
    )(degp)


def _tc_finish(part, dinv, W, a1):
    BR = 1000
    grid = (N_NODES // BR,)

    def body(p0_r, p1_r, d_r, w_r, a_r, o_r):
        x = p0_r[0] + p1_r[0]
        y = lax.dot_general(x, w_r[...], (((1,), (1,)), ((), ())),
                            preferred_element_type=jnp.float32)
        y = y * d_r[...]
        a = a_r[0]
        o_r[...] = jnp.where(y >= 0.0, y, a * y)

    return pl.pallas_call(
        body,
        grid=grid,
        in_specs=[
            pl.BlockSpec((1, BR, DIM), lambda i: (0, i, 0)),
            pl.BlockSpec((1, BR, DIM), lambda i: (1, i, 0)),
            pl.BlockSpec((BR, 1), lambda i: (i, 0)),
            pl.BlockSpec((DIM, DIM), lambda i: (0, 0)),
            pl.BlockSpec(memory_space=pltpu.SMEM),
        ],
        out_specs=pl.BlockSpec((BR, DIM), lambda i: (i, 0)),
        out_shape=jax.ShapeDtypeStruct((N_NODES, DIM), jnp.float32),
    )(part, part, dinv, W, a1)


def kernel(seq, edge_index, W, prelu_a):
    edge_flat = edge_index.astype(jnp.int32).reshape(2 * N_EDGES)
    part, degp = _sc_aggregate(seq, edge_flat)
    dinv = _tc_deg_reduce(degp).reshape(NP)[:N_NODES].reshape(N_NODES, 1)
    a1 = prelu_a.reshape(1)
    return _tc_finish(part, dinv, W, a1)


# speedup vs baseline: 1.1780x; 1.0197x over previous
import functools

import jax
import jax.numpy as jnp
from jax import lax
from jax.experimental import pallas as pl
from jax.experimental.pallas import tpu as pltpu
from jax.experimental.pallas import tpu_sc as plsc

N_NODES = 10000
N_EDGES = 320000
DIM = 128
NP = 10240
CHUNK = 128
NC, NS = 2, 16
NW = NC * NS
N_CHUNKS = N_EDGES // CHUNK
UNROLL = 6
NROWS = 2
NIDX = 6
NITER = 13


def _sc_aggregate(seq, edge_flat):
    mesh = plsc.VectorSubcoreMesh(core_axis_name="c", subcore_axis_name="s")

    scratch = [
        pltpu.VMEM_SHARED((NP, DIM), jnp.float32),
        pltpu.VMEM_SHARED((NP,), jnp.float32),
        pltpu.VMEM((CHUNK,), jnp.float32),
        pltpu.VMEM((640,), jnp.float32),
        pltpu.VMEM((64, DIM), jnp.float32),
        pltpu.VMEM((8, DIM), jnp.float32),
    ]
    scratch += [pltpu.VMEM((CHUNK,), jnp.int32) for _ in range(NIDX)]
    scratch += [pltpu.VMEM((CHUNK,), jnp.int32) for _ in range(NIDX)]
    scratch += [pltpu.VMEM((CHUNK, DIM), jnp.float32) for _ in range(NROWS)]
    scratch += [pltpu.SemaphoreType.DMA for _ in range(2 * NIDX + NROWS)]
    scratch += [pltpu.SemaphoreType.DMA]

    @functools.partial(
        pl.kernel,
        out_type=(
            jax.ShapeDtypeStruct((NC, NP, DIM), jnp.float32),
            jax.ShapeDtypeStruct((NC, 80, DIM), jnp.float32),
        ),
        mesh=mesh,
        scratch_types=scratch,
    )
    def sc_agg(seq_hbm, edge_hbm, part_hbm, degp_hbm,
               acc_sh, deg_sh, ones_v, zero1_v, zero_v, dbounce_v, *bufs):
        src_vs = bufs[0:NIDX]
        dst_vs = bufs[NIDX:2 * NIDX]
        rows_vs = bufs[2 * NIDX:2 * NIDX + NROWS]
        isems = bufs[2 * NIDX + NROWS:3 * NIDX + NROWS]
        gsems = bufs[3 * NIDX + NROWS:3 * NIDX + 2 * NROWS]
        dsems = bufs[3 * NIDX + 2 * NROWS:4 * NIDX + 2 * NROWS]
        zsem = bufs[4 * NIDX + 2 * NROWS]

        cid = lax.axis_index("c")
        sid = lax.axis_index("s")
        wid = cid * NS + sid
        zvec = jnp.zeros((16,), jnp.float32)

        def zb(i, _):
            zero_v[i // 8, pl.ds((i % 8) * 16, 16)] = zvec
            return 0
        lax.fori_loop(0, 64 * 8, zb, 0)

        def z1(i, _):
            zero1_v[pl.ds(i * 16, 16)] = zvec
            return 0
        lax.fori_loop(0, 40, z1, 0)

        for j in range(CHUNK // 16):
            ones_v[pl.ds(j * 16, 16)] = jnp.ones((16,), jnp.float32)

        zdescs = [
            pltpu.async_copy(zero_v, acc_sh.at[pl.ds(sid * 640 + t * 64, 64)],
                             zsem)
            for t in range(10)
        ]
        zdescs.append(
            pltpu.async_copy(zero1_v, deg_sh.at[pl.ds(sid * 640, 640)], zsem))

        N_MAIN = NITER * UNROLL

        def issue_idx(v, s):
            base = pl.multiple_of((wid + v * NW) * CHUNK, CHUNK)
            pltpu.async_copy(edge_hbm.at[pl.ds(N_EDGES + base, CHUNK)],
                             src_vs[s], isems[s])
            pltpu.async_copy(edge_hbm.at[pl.ds(base, CHUNK)],
                             dst_vs[s], isems[s])

        def wait_idx(s):
            pltpu.make_async_copy(edge_hbm.at[pl.ds(0, CHUNK)], src_vs[s],
                                  isems[s]).wait()
            pltpu.make_async_copy(edge_hbm.at[pl.ds(0, CHUNK)], dst_vs[s],
                                  isems[s]).wait()

        def issue_gather(s, b):
            pltpu.async_copy(seq_hbm.at[src_vs[s]], rows_vs[b], gsems[b])

        def wait_gather(s, b):
            pltpu.make_async_copy(seq_hbm.at[src_vs[s]], rows_vs[b],
                                  gsems[b]).wait()

        def wait_deg(s):
            pltpu.make_async_copy(ones_v, deg_sh.at[dst_vs[s]],
                                  dsems[s]).wait()

        for v in range(4):
            issue_idx(v, v % NIDX)
        wait_idx(0)
        issue_gather(0, 0)
        wait_idx(1)
        issue_gather(1, 1)

        for zd in zdescs:
            zd.wait()
        plsc.subcore_barrier()

        def body(t, _):
            c0 = t * UNROLL
            for u in range(UNROLL):
                b = u % NROWS
                wait_gather(u, b)
                pltpu.sync_copy(rows_vs[b], acc_sh.at[dst_vs[u]], add=True)
                pltpu.async_copy(ones_v, deg_sh.at[dst_vs[u]], dsems[u],
                                 add=True)

                @pl.when(c0 + u >= 2)
                def _(u=u):
                    wait_deg((u - 2) % NIDX)

                @pl.when(c0 + u + 4 < N_MAIN)
                def _(u=u, c0=c0):
                    issue_idx(c0 + u + 4, (u + 4) % NIDX)

                @pl.when(c0 + u + 2 < N_MAIN)
                def _(u=u, b=b):
                    s = (u + 2) % NIDX
                    wait_idx(s)
                    issue_gather(s, b)
            return 0
        lax.fori_loop(0, NITER, body, 0)

        wait_deg((N_MAIN - 2) % NIDX)
        wait_deg((N_MAIN - 1) % NIDX)

        @pl.when(wid < N_CHUNKS - NITER * UNROLL * NW)
        def _():
            base = pl.multiple_of((NITER * UNROLL * NW + wid) * CHUNK, CHUNK)
            pltpu.sync_copy(edge_hbm.at[pl.ds(N_EDGES + base, CHUNK)],
                            src_vs[0])
            pltpu.sync_copy(edge_hbm.at[pl.ds(base, CHUNK)], dst_vs[0])
            pltpu.async_copy(seq_hbm.at[src_vs[0]], rows_vs[0],
                             gsems[0]).wait()
            pltpu.sync_copy(rows_vs[0], acc_sh.at[dst_vs[0]], add=True)
            pltpu.sync_copy(ones_v, deg_sh.at[dst_vs[0]], add=True)

        plsc.subcore_barrier()

        cdesc = pltpu.async_copy(acc_sh.at[pl.ds(sid * 640, 640)],
                                 part_hbm.at[cid, pl.ds(sid * 640, 640)],
                                 zsem)

        @pl.when(sid < 10)
        def _():
            bdescs = [
                pltpu.async_copy(
                    deg_sh.at[pl.ds((sid * 8 + q) * DIM, DIM)],
                    dbounce_v.at[q], zsem)
                for q in range(8)
            ]
            for bd in bdescs:
                bd.wait()
            pltpu.sync_copy(dbounce_v, degp_hbm.at[cid, pl.ds(sid * 8, 8)])

        cdesc.wait()

    return sc_agg(seq, edge_flat)


def _tc_deg_reduce(degp):
    def body(d_r, o_r):
        o_r[...] = 1.0 / (d_r[0] + d_r[1] + 1e-8)

    return pl.pallas_call(
        body,
        out_shape=jax.ShapeDtypeStruct((80, DIM), jnp.float32),
    )(degp)


def _tc_finish(part, dinv, W, a1):
    BR = 2000
    grid = (N_NODES // BR,)

    def body(p0_r, p1_r, d_r, w_r, a_r, o_r):
        x = p0_r[0] + p1_r[0]
        y = lax.dot_general(x, w_r[...], (((1,), (1,)), ((), ())),
                            preferred_element_type=jnp.float32)
        y = y * d_r[...]
        a = a_r[0]
        o_r[...] = jnp.where(y >= 0.0, y, a * y)

    return pl.pallas_call(
        body,
        grid=grid,
        in_specs=[
            pl.BlockSpec((1, BR, DIM), lambda i: (0, i, 0)),
            pl.BlockSpec((1, BR, DIM), lambda i: (1, i, 0)),
            pl.BlockSpec((BR, 1), lambda i: (i, 0)),
            pl.BlockSpec((DIM, DIM), lambda i: (0, 0)),
            pl.BlockSpec(memory_space=pltpu.SMEM),
        ],
        out_specs=pl.BlockSpec((BR, DIM), lambda i: (i, 0)),
        out_shape=jax.ShapeDtypeStruct((N_NODES, DIM), jnp.float32),
    )(part, part, dinv, W, a1)


def kernel(seq, edge_index, W, prelu_a):
    edge_flat = edge_index.astype(jnp.int32).reshape(2 * N_EDGES)
    part, degp = _sc_aggregate(seq, edge_flat)
    dinv = _tc_deg_reduce(degp).reshape(NP)[:N_NODES].reshape(N_NODES, 1)
    a1 = prelu_a.reshape(1)
    return _tc_finish(part, dinv, W, a1)
